# baseline (speedup 1.0000x reference)
import jax
import jax.numpy as jnp
from jax.experimental import pallas as pl
from jax.experimental.pallas import tpu as pltpu

_E = 8
_IN_DIM = 2048
_B = 2048
_HIDDEN = 256
_D_OUT = 64

_NB = 1
_BT = _B // _NB
_CHUNK = 256


def _moe_body(x_ref, W0_ref, b0_ref, W1_ref, b1_ref, Wmulv_ref, bmulv_ref,
              V0p_ref, c0_ref, V1_ref, c1_ref,
              Vout_ref, cout_ref,
              mulv_out, xhat_out, idx_out, best_ref):
    e = pl.program_id(1)
    W0 = W0_ref[0]
    W1 = W1_ref[0]
    Wmulv = Wmulv_ref[0]
    V0p = V0p_ref[0]
    V1 = V1_ref[0]
    Vout = Vout_ref[0]
    b0 = b0_ref[0]
    b1 = b1_ref[0]
    bmulv = bmulv_ref[0]
    c0 = c0_ref[0]
    c1 = c1_ref[0]
    cout = cout_ref[0]

    for c in range(_BT // _CHUNK):
        sl = pl.ds(c * _CHUNK, _CHUNK)
        x = x_ref[sl, :]
        h = jnp.maximum(jnp.dot(x, W0, preferred_element_type=jnp.float32) + b0, 0.0)
        h = jnp.maximum(jnp.dot(h, W1, preferred_element_type=jnp.float32) + b1, 0.0)
        mulv = jnp.dot(h, Wmulv, preferred_element_type=jnp.float32) + bmulv
        g = jnp.maximum(jnp.dot(mulv, V0p, preferred_element_type=jnp.float32) + c0, 0.0)
        g = jnp.maximum(jnp.dot(g, V1, preferred_element_type=jnp.float32) + c1, 0.0)
        xh = jnp.dot(g, Vout, preferred_element_type=jnp.float32) + cout
        d = xh - x
        loss = jnp.mean(d * d, axis=1, keepdims=True)

        @pl.when(e == 0)
        def _():
            best_ref[sl, :] = jnp.full((_CHUNK, 1), jnp.inf, jnp.float32)

        mask = loss < best_ref[sl, :]
        best_ref[sl, :] = jnp.where(mask, loss, best_ref[sl, :])
        mulv_out[sl, :] = jnp.where(mask, mulv, mulv_out[sl, :])
        xhat_out[sl, :] = jnp.where(mask, xh, xhat_out[sl, :])
        idx_out[sl, :] = jnp.where(mask, e, idx_out[sl, :])


def kernel(x, params):
    p = params
    grid = (_NB, _E)

    def wspec(shape):
        return pl.BlockSpec((1,) + shape, lambda i, e: (e,) + (0,) * len(shape))

    in_specs = [
        pl.BlockSpec((_BT, _IN_DIM), lambda i, e: (i, 0)),
        wspec((_IN_DIM, _HIDDEN)),
        wspec((1, _HIDDEN)),
        wspec((_HIDDEN, _HIDDEN)),
        wspec((1, _HIDDEN)),
        wspec((_HIDDEN, 2 * _D_OUT)),
        wspec((1, 2 * _D_OUT)),
        wspec((2 * _D_OUT, _HIDDEN)),
        wspec((1, _HIDDEN)),
        wspec((_HIDDEN, _HIDDEN)),
        wspec((1, _HIDDEN)),
        wspec((_HIDDEN, _IN_DIM)),
        wspec((1, _IN_DIM)),
    ]
    out_specs = [
        pl.BlockSpec((_BT, 2 * _D_OUT), lambda i, e: (i, 0)),
        pl.BlockSpec((_BT, _IN_DIM), lambda i, e: (i, 0)),
        pl.BlockSpec((_BT, 1), lambda i, e: (i, 0)),
    ]
    out_shape = [
        jax.ShapeDtypeStruct((_B, 2 * _D_OUT), jnp.float32),
        jax.ShapeDtypeStruct((_B, _IN_DIM), jnp.float32),
        jax.ShapeDtypeStruct((_B, 1), jnp.int32),
    ]

    Wmulv = jnp.concatenate([p["Wmu"], p["Wlv"]], axis=2)
    bmulv = jnp.concatenate([p["bmu"], p["blv"]], axis=1)[:, None, :]
    V0p = jnp.concatenate(
        [p["V0"], jnp.zeros_like(p["V0"])], axis=1)

    mulv_sel, xhat_sel, idx = pl.pallas_call(
        _moe_body,
        grid=grid,
        in_specs=in_specs,
        out_specs=out_specs,
        out_shape=out_shape,
        scratch_shapes=[pltpu.VMEM((_BT, 1), jnp.float32)],
        compiler_params=pltpu.CompilerParams(
            dimension_semantics=("parallel", "arbitrary")),
    )(x,
      p["W0"], p["b0"][:, None, :], p["W1"], p["b1"][:, None, :],
      Wmulv, bmulv, V0p, p["c0"][:, None, :],
      p["V1"], p["c1"][:, None, :],
      p["Vout"], p["cout"][:, None, :])

    return (mulv_sel[:, :_D_OUT], mulv_sel[:, _D_OUT:], xhat_sel, idx[:, 0])


# speedup vs baseline: 1.2121x; 1.2121x over previous
import jax
import jax.numpy as jnp
from jax.experimental import pallas as pl
from jax.experimental.pallas import tpu as pltpu

_E = 8
_IN_DIM = 2048
_B = 2048
_HIDDEN = 256
_D_OUT = 64

_NB = 2
_BT = _B // _NB
_CHUNK = 256


def _moe_body(x_ref, W0_ref, b0_ref, W1_ref, b1_ref, Wmulv_ref, bmulv_ref,
              V0p_ref, c0_ref, V1_ref, c1_ref,
              Vout_ref, cout_ref,
              mulv_out, xhat_out, idx_out, best_ref):
    e = pl.program_id(1)
    W0 = W0_ref[0]
    W1 = W1_ref[0]
    Wmulv = Wmulv_ref[0]
    V0p = V0p_ref[0]
    V1 = V1_ref[0]
    Vout = Vout_ref[0]
    b0 = b0_ref[0]
    b1 = b1_ref[0]
    bmulv = bmulv_ref[0]
    c0 = c0_ref[0]
    c1 = c1_ref[0]
    cout = cout_ref[0]

    h = jnp.maximum(jnp.dot(x_ref[...], W0, preferred_element_type=jnp.float32) + b0, 0.0)
    h = jnp.maximum(jnp.dot(h, W1, preferred_element_type=jnp.float32) + b1, 0.0)
    mulv = jnp.dot(h, Wmulv, preferred_element_type=jnp.float32) + bmulv
    g = jnp.maximum(jnp.dot(mulv, V0p, preferred_element_type=jnp.float32) + c0, 0.0)
    g = jnp.maximum(jnp.dot(g, V1, preferred_element_type=jnp.float32) + c1, 0.0)

    @pl.when(e == 0)
    def _():
        best_ref[...] = jnp.full((_BT, 1), jnp.inf, jnp.float32)

    for c in range(_BT // _CHUNK):
        sl = pl.ds(c * _CHUNK, _CHUNK)
        xh = jnp.dot(g[c * _CHUNK:(c + 1) * _CHUNK],
                     Vout, preferred_element_type=jnp.float32) + cout
        d = xh - x_ref[sl, :]
        loss = jnp.mean(d * d, axis=1, keepdims=True)

        mask = loss < best_ref[sl, :]
        best_ref[sl, :] = jnp.where(mask, loss, best_ref[sl, :])
        mulv_out[sl, :] = jnp.where(
            mask, mulv[c * _CHUNK:(c + 1) * _CHUNK], mulv_out[sl, :])
        xhat_out[sl, :] = jnp.where(mask, xh, xhat_out[sl, :])
        idx_out[sl, :] = jnp.where(mask, e, idx_out[sl, :])


def kernel(x, params):
    p = params
    grid = (_NB, _E)

    def wspec(shape):
        return pl.BlockSpec((1,) + shape, lambda i, e: (e,) + (0,) * len(shape))

    in_specs = [
        pl.BlockSpec((_BT, _IN_DIM), lambda i, e: (i, 0)),
        wspec((_IN_DIM, _HIDDEN)),
        wspec((1, _HIDDEN)),
        wspec((_HIDDEN, _HIDDEN)),
        wspec((1, _HIDDEN)),
        wspec((_HIDDEN, 2 * _D_OUT)),
        wspec((1, 2 * _D_OUT)),
        wspec((2 * _D_OUT, _HIDDEN)),
        wspec((1, _HIDDEN)),
        wspec((_HIDDEN, _HIDDEN)),
        wspec((1, _HIDDEN)),
        wspec((_HIDDEN, _IN_DIM)),
        wspec((1, _IN_DIM)),
    ]
    out_specs = [
        pl.BlockSpec((_BT, 2 * _D_OUT), lambda i, e: (i, 0)),
        pl.BlockSpec((_BT, _IN_DIM), lambda i, e: (i, 0)),
        pl.BlockSpec((_BT, 1), lambda i, e: (i, 0)),
    ]
    out_shape = [
        jax.ShapeDtypeStruct((_B, 2 * _D_OUT), jnp.float32),
        jax.ShapeDtypeStruct((_B, _IN_DIM), jnp.float32),
        jax.ShapeDtypeStruct((_B, 1), jnp.int32),
    ]

    Wmulv = jnp.concatenate([p["Wmu"], p["Wlv"]], axis=2)
    bmulv = jnp.concatenate([p["bmu"], p["blv"]], axis=1)[:, None, :]
    V0p = jnp.concatenate(
        [p["V0"], jnp.zeros_like(p["V0"])], axis=1)

    mulv_sel, xhat_sel, idx = pl.pallas_call(
        _moe_body,
        grid=grid,
        in_specs=in_specs,
        out_specs=out_specs,
        out_shape=out_shape,
        scratch_shapes=[pltpu.VMEM((_BT, 1), jnp.float32)],
        compiler_params=pltpu.CompilerParams(
            dimension_semantics=("parallel", "arbitrary")),
    )(x,
      p["W0"], p["b0"][:, None, :], p["W1"], p["b1"][:, None, :],
      Wmulv, bmulv, V0p, p["c0"][:, None, :],
      p["V1"], p["c1"][:, None, :],
      p["Vout"], p["cout"][:, None, :])

    return (mulv_sel[:, :_D_OUT], mulv_sel[:, _D_OUT:], xhat_sel, idx[:, 0])
